# Initial kernel scaffold; baseline (speedup 1.0000x reference)
import jax
import jax.numpy as jnp
from jax.experimental import pallas as pl


def kernel(user_indices, item_indices, emb_user_mlp, emb_item_mlp, emb_user_mf, emb_item_mf, W0, b0, W1, b1, W2, b2, Wa, ba):
    raise NotImplementedError("write your pallas kernel here")


# speedup vs baseline: 2.5652x; 2.5652x over previous
import functools

import jax
import jax.numpy as jnp
from jax import lax
from jax.experimental import pallas as pl
from jax.experimental.pallas import tpu as pltpu
from jax.experimental.pallas import tpu_sc as plsc

B = 4096
L = 50
BL = B * L
D = 16
NW = 32
ROWS_W = BL // NW
USERS_W = B // NW
CSZ = 1600
NCHUNK = ROWS_W // CSZ


def _sc_gather(item_idx, user_idx, t_imlp, t_imf, t_umlp, t_umf):
    f32 = jnp.float32
    mesh = plsc.VectorSubcoreMesh(core_axis_name="c", subcore_axis_name="s")

    @functools.partial(
        pl.kernel,
        mesh=mesh,
        out_type=[
            jax.ShapeDtypeStruct((BL, D), f32),
            jax.ShapeDtypeStruct((BL, D), f32),
            jax.ShapeDtypeStruct((B, D), f32),
            jax.ShapeDtypeStruct((B, D), f32),
        ],
        scratch_types=[
            pltpu.VMEM((CSZ,), jnp.int32),
            pltpu.VMEM((CSZ, D), f32),
            pltpu.VMEM((USERS_W,), jnp.int32),
            pltpu.VMEM((USERS_W, D), f32),
            pltpu.SemaphoreType.DMA,
        ],
        compiler_params=pltpu.CompilerParams(use_tc_tiling_on_sc=False),
    )
    def gather(item_idx_hbm, user_idx_hbm, timlp, timf, tumlp, tumf,
               o_imlp, o_imf, o_umlp, o_umf,
               idx_v, rows_v, uidx_v, urows_v, sem):
        wid = lax.axis_index("s") * 2 + lax.axis_index("c")
        base = wid * ROWS_W
        for tbl, out in ((timlp, o_imlp), (timf, o_imf)):
            for k in range(NCHUNK):
                off = base + k * CSZ
                pltpu.sync_copy(item_idx_hbm.at[pl.ds(off, CSZ)], idx_v)
                pltpu.async_copy(tbl.at[idx_v], rows_v, sem).wait()
                pltpu.sync_copy(rows_v, out.at[pl.ds(off, CSZ)])
        ubase = wid * USERS_W
        pltpu.sync_copy(user_idx_hbm.at[pl.ds(ubase, USERS_W)], uidx_v)
        for tbl, out in ((tumlp, o_umlp), (tumf, o_umf)):
            pltpu.async_copy(tbl.at[uidx_v], urows_v, sem).wait()
            pltpu.sync_copy(urows_v, out.at[pl.ds(ubase, USERS_W)])

    return gather(item_idx, user_idx, t_imlp, t_imf, t_umlp, t_umf)


BB = 128
R = BB * L


def _tc_body(ie_mlp_ref, ie_mf_ref, ue_mlp_ref, ue_mf_ref,
             w0u_ref, w0i_ref, w1_ref, w2_ref, b0_ref, b1_ref, b2_ref,
             wam_ref, waf_ref, ba_ref, out_ref):
    f32 = jnp.float32
    r_iota = lax.broadcasted_iota(jnp.int32, (R, BB), 0)
    c_iota = lax.broadcasted_iota(jnp.int32, (R, BB), 1)
    sel = jnp.where(r_iota // L == c_iota, 1.0, 0.0).astype(f32)
    u0 = jnp.dot(ue_mlp_ref[...], w0u_ref[...], preferred_element_type=f32) + b0_ref[...]
    h = jnp.dot(ie_mlp_ref[...], w0i_ref[...], preferred_element_type=f32)
    h = jnp.maximum(h + jnp.dot(sel, u0, preferred_element_type=f32), 0.0)
    h = jnp.maximum(jnp.dot(h, w1_ref[...], preferred_element_type=f32) + b1_ref[...], 0.0)
    h = jnp.maximum(jnp.dot(h, w2_ref[...], preferred_element_type=f32) + b2_ref[...], 0.0)
    mf = ie_mf_ref[...] * jnp.dot(sel, ue_mf_ref[...], preferred_element_type=f32)
    out_ref[...] = (jnp.dot(h, wam_ref[...], preferred_element_type=f32)
                    + jnp.dot(mf, waf_ref[...], preferred_element_type=f32)
                    + ba_ref[...])


def _tc_mlp(ie_mlp, ie_mf, ue_mlp, ue_mf, w0u, w0i, w1, w2, b0, b1, b2,
            wam, waf, ba):
    full = lambda shape: pl.BlockSpec(shape, lambda i: (0, 0))
    return pl.pallas_call(
        _tc_body,
        grid=(B // BB,),
        in_specs=[
            pl.BlockSpec((R, D), lambda i: (i, 0)),
            pl.BlockSpec((R, D), lambda i: (i, 0)),
            pl.BlockSpec((BB, D), lambda i: (i, 0)),
            pl.BlockSpec((BB, D), lambda i: (i, 0)),
            full((D, 64)),
            full((D, 64)),
            full((64, 32)),
            full((32, D)),
            full((1, 64)),
            full((1, 32)),
            full((1, D)),
            full((D, 1)),
            full((D, 1)),
            full((1, 1)),
        ],
        out_specs=pl.BlockSpec((R, 1), lambda i: (i, 0)),
        out_shape=jax.ShapeDtypeStruct((BL, 1), jnp.float32),
    )(ie_mlp, ie_mf, ue_mlp, ue_mf, w0u, w0i, w1, w2, b0, b1, b2, wam, waf, ba)


def kernel(user_indices, item_indices, emb_user_mlp, emb_item_mlp,
           emb_user_mf, emb_item_mf, W0, b0, W1, b1, W2, b2, Wa, ba):
    item_flat = item_indices.reshape(-1).astype(jnp.int32)
    user_idx = user_indices.astype(jnp.int32)
    ie_mlp, ie_mf, ue_mlp, ue_mf = _sc_gather(
        item_flat, user_idx, emb_item_mlp, emb_item_mf, emb_user_mlp, emb_user_mf)
    out = _tc_mlp(
        ie_mlp, ie_mf, ue_mlp, ue_mf,
        W0[:D], W0[D:], W1, W2,
        b0[None, :], b1[None, :], b2[None, :],
        Wa[:D], Wa[D:], ba[None, :])
    return out.reshape(B, 1, L)


# speedup vs baseline: 2.9464x; 1.1486x over previous
import functools

import jax
import jax.numpy as jnp
from jax import lax
from jax.experimental import pallas as pl
from jax.experimental.pallas import tpu as pltpu
from jax.experimental.pallas import tpu_sc as plsc

B = 4096
L = 50
BL = B * L
D = 16
NW = 32
ROWS_W = BL // NW
USERS_W = B // NW
CSZ = 1600
NCHUNK = ROWS_W // CSZ

RP_CB = 8192
RP_GRID = 123
NPACK = RP_GRID * RP_CB


def _rp_body(in_ref, out_ref):
    y = jnp.transpose(in_ref[...])
    w = y.reshape(RP_CB // 8, 8, D)
    out_ref[...] = jnp.concatenate([w[:, g, :] for g in range(8)], axis=1)


def _tc_repack(embT):
    return pl.pallas_call(
        _rp_body,
        grid=(RP_GRID,),
        in_specs=[pl.BlockSpec((D, RP_CB), lambda i: (0, i))],
        out_specs=pl.BlockSpec((RP_CB // 8, 8 * D), lambda i: (i, 0)),
        out_shape=jax.ShapeDtypeStruct((NPACK // 8, 8 * D), jnp.float32),
    )(embT).reshape(NPACK, D)


def _sc_gather(item_idx, user_idx, t_imlp, t_imf, t_umlp, t_umf):
    f32 = jnp.float32
    mesh = plsc.VectorSubcoreMesh(core_axis_name="c", subcore_axis_name="s")

    @functools.partial(
        pl.kernel,
        mesh=mesh,
        out_type=[
            jax.ShapeDtypeStruct((BL, D), f32),
            jax.ShapeDtypeStruct((BL, D), f32),
            jax.ShapeDtypeStruct((B, D), f32),
            jax.ShapeDtypeStruct((B, D), f32),
        ],
        scratch_types=[
            pltpu.VMEM((CSZ,), jnp.int32),
            pltpu.VMEM((CSZ, D), f32),
            pltpu.VMEM((USERS_W,), jnp.int32),
            pltpu.VMEM((USERS_W, D), f32),
            pltpu.SemaphoreType.DMA,
        ],
        compiler_params=pltpu.CompilerParams(use_tc_tiling_on_sc=False),
    )
    def gather(item_idx_hbm, user_idx_hbm, timlp, timf, tumlp, tumf,
               o_imlp, o_imf, o_umlp, o_umf,
               idx_v, rows_v, uidx_v, urows_v, sem):
        wid = lax.axis_index("s") * 2 + lax.axis_index("c")
        base = wid * ROWS_W
        for tbl, out in ((timlp, o_imlp), (timf, o_imf)):
            for k in range(NCHUNK):
                off = base + k * CSZ
                pltpu.sync_copy(item_idx_hbm.at[pl.ds(off, CSZ)], idx_v)
                pltpu.async_copy(tbl.at[idx_v], rows_v, sem).wait()
                pltpu.sync_copy(rows_v, out.at[pl.ds(off, CSZ)])
        ubase = wid * USERS_W
        pltpu.sync_copy(user_idx_hbm.at[pl.ds(ubase, USERS_W)], uidx_v)
        for tbl, out in ((tumlp, o_umlp), (tumf, o_umf)):
            pltpu.async_copy(tbl.at[uidx_v], urows_v, sem).wait()
            pltpu.sync_copy(urows_v, out.at[pl.ds(ubase, USERS_W)])

    return gather(item_idx, user_idx, t_imlp, t_imf, t_umlp, t_umf)


BB = 128
R = BB * L


def _tc_body(ie_mlp_ref, ie_mf_ref, ue_mlp_ref, ue_mf_ref,
             w0u_ref, w0i_ref, w1_ref, w2_ref, b0_ref, b1_ref, b2_ref,
             wam_ref, waf_ref, ba_ref, out_ref):
    f32 = jnp.float32
    r_iota = lax.broadcasted_iota(jnp.int32, (R, BB), 0)
    c_iota = lax.broadcasted_iota(jnp.int32, (R, BB), 1)
    sel = jnp.where(r_iota // L == c_iota, 1.0, 0.0).astype(f32)
    u0 = jnp.dot(ue_mlp_ref[...], w0u_ref[...], preferred_element_type=f32) + b0_ref[...]
    h = jnp.dot(ie_mlp_ref[...], w0i_ref[...], preferred_element_type=f32)
    h = jnp.maximum(h + jnp.dot(sel, u0, preferred_element_type=f32), 0.0)
    h = jnp.maximum(jnp.dot(h, w1_ref[...], preferred_element_type=f32) + b1_ref[...], 0.0)
    h = jnp.maximum(jnp.dot(h, w2_ref[...], preferred_element_type=f32) + b2_ref[...], 0.0)
    mf = ie_mf_ref[...] * jnp.dot(sel, ue_mf_ref[...], preferred_element_type=f32)
    out_ref[...] = (jnp.dot(h, wam_ref[...], preferred_element_type=f32)
                    + jnp.dot(mf, waf_ref[...], preferred_element_type=f32)
                    + ba_ref[...])


def _tc_mlp(ie_mlp, ie_mf, ue_mlp, ue_mf, w0u, w0i, w1, w2, b0, b1, b2,
            wam, waf, ba):
    full = lambda shape: pl.BlockSpec(shape, lambda i: (0, 0))
    return pl.pallas_call(
        _tc_body,
        grid=(B // BB,),
        in_specs=[
            pl.BlockSpec((R, D), lambda i: (i, 0)),
            pl.BlockSpec((R, D), lambda i: (i, 0)),
            pl.BlockSpec((BB, D), lambda i: (i, 0)),
            pl.BlockSpec((BB, D), lambda i: (i, 0)),
            full((D, 64)),
            full((D, 64)),
            full((64, 32)),
            full((32, D)),
            full((1, 64)),
            full((1, 32)),
            full((1, D)),
            full((D, 1)),
            full((D, 1)),
            full((1, 1)),
        ],
        out_specs=pl.BlockSpec((R, 1), lambda i: (i, 0)),
        out_shape=jax.ShapeDtypeStruct((BL, 1), jnp.float32),
    )(ie_mlp, ie_mf, ue_mlp, ue_mf, w0u, w0i, w1, w2, b0, b1, b2, wam, waf, ba)


def kernel(user_indices, item_indices, emb_user_mlp, emb_item_mlp,
           emb_user_mf, emb_item_mf, W0, b0, W1, b1, W2, b2, Wa, ba):
    item_flat = item_indices.reshape(-1).astype(jnp.int32)
    user_idx = user_indices.astype(jnp.int32)
    g_imlp = _tc_repack(emb_item_mlp.T)
    g_imf = _tc_repack(emb_item_mf.T)
    g_umlp = _tc_repack(emb_user_mlp.T)
    g_umf = _tc_repack(emb_user_mf.T)
    ie_mlp, ie_mf, ue_mlp, ue_mf = _sc_gather(
        item_flat, user_idx, g_imlp, g_imf, g_umlp, g_umf)
    out = _tc_mlp(
        ie_mlp, ie_mf, ue_mlp, ue_mf,
        W0[:D], W0[D:], W1, W2,
        b0[None, :], b1[None, :], b2[None, :],
        Wa[:D], Wa[D:], ba[None, :])
    return out.reshape(B, 1, L)


# speedup vs baseline: 6.1766x; 2.0963x over previous
import functools

import jax
import jax.numpy as jnp
from jax import lax
from jax.experimental import pallas as pl
from jax.experimental.pallas import tpu as pltpu
from jax.experimental.pallas import tpu_sc as plsc

B = 4096
L = 50
BL = B * L
D = 16
NW = 32
ROWS_W = BL // NW
USERS_W = B // NW
CSZ = 1600
NCHUNK = ROWS_W // CSZ

RP_CB = 8192
RP_GRID = 123
NPACK = RP_GRID * RP_CB


def _rp_body(in_ref, out_ref):
    x = in_ref[...]
    x8 = jnp.concatenate(
        [x[:, 1024 * a:1024 * (a + 1)] for a in range(8)], axis=0)
    out_ref[...] = jnp.transpose(x8)


def _tc_repack(embT):
    return pl.pallas_call(
        _rp_body,
        grid=(RP_GRID,),
        in_specs=[pl.BlockSpec((D, RP_CB), lambda i: (0, i))],
        out_specs=pl.BlockSpec((RP_CB // 8, 8 * D), lambda i: (i, 0)),
        out_shape=jax.ShapeDtypeStruct((NPACK // 8, 8 * D), jnp.float32),
    )(embT).reshape(NPACK, D)


def _permute_idx(r):
    i = r >> 13
    m = r & 8191
    return (i << 13) + ((m & 1023) << 3) + (m >> 10)


def _sc_gather(item_idx, user_idx, t_imlp, t_imf, t_umlp, t_umf):
    f32 = jnp.float32
    mesh = plsc.VectorSubcoreMesh(core_axis_name="c", subcore_axis_name="s")

    @functools.partial(
        pl.kernel,
        mesh=mesh,
        out_type=[
            jax.ShapeDtypeStruct((BL, D), f32),
            jax.ShapeDtypeStruct((BL, D), f32),
            jax.ShapeDtypeStruct((B, D), f32),
            jax.ShapeDtypeStruct((B, D), f32),
        ],
        scratch_types=[
            pltpu.VMEM((CSZ,), jnp.int32),
            pltpu.VMEM((CSZ, D), f32),
            pltpu.VMEM((USERS_W,), jnp.int32),
            pltpu.VMEM((USERS_W, D), f32),
            pltpu.SemaphoreType.DMA,
        ],
        compiler_params=pltpu.CompilerParams(use_tc_tiling_on_sc=False),
    )
    def gather(item_idx_hbm, user_idx_hbm, timlp, timf, tumlp, tumf,
               o_imlp, o_imf, o_umlp, o_umf,
               idx_v, rows_v, uidx_v, urows_v, sem):
        wid = lax.axis_index("s") * 2 + lax.axis_index("c")
        base = wid * ROWS_W
        for tbl, out in ((timlp, o_imlp), (timf, o_imf)):
            for k in range(NCHUNK):
                off = base + k * CSZ
                pltpu.sync_copy(item_idx_hbm.at[pl.ds(off, CSZ)], idx_v)
                pltpu.async_copy(tbl.at[idx_v], rows_v, sem).wait()
                pltpu.sync_copy(rows_v, out.at[pl.ds(off, CSZ)])
        ubase = wid * USERS_W
        pltpu.sync_copy(user_idx_hbm.at[pl.ds(ubase, USERS_W)], uidx_v)
        for tbl, out in ((tumlp, o_umlp), (tumf, o_umf)):
            pltpu.async_copy(tbl.at[uidx_v], urows_v, sem).wait()
            pltpu.sync_copy(urows_v, out.at[pl.ds(ubase, USERS_W)])

    return gather(item_idx, user_idx, t_imlp, t_imf, t_umlp, t_umf)


BB = 128
R = BB * L


def _tc_body(ie_mlp_ref, ie_mf_ref, ue_mlp_ref, ue_mf_ref,
             w0u_ref, w0i_ref, w1_ref, w2_ref, b0_ref, b1_ref, b2_ref,
             wam_ref, waf_ref, ba_ref, out_ref):
    f32 = jnp.float32
    r_iota = lax.broadcasted_iota(jnp.int32, (R, BB), 0)
    c_iota = lax.broadcasted_iota(jnp.int32, (R, BB), 1)
    sel = jnp.where(r_iota // L == c_iota, 1.0, 0.0).astype(f32)
    u0 = jnp.dot(ue_mlp_ref[...], w0u_ref[...], preferred_element_type=f32) + b0_ref[...]
    h = jnp.dot(ie_mlp_ref[...], w0i_ref[...], preferred_element_type=f32)
    h = jnp.maximum(h + jnp.dot(sel, u0, preferred_element_type=f32), 0.0)
    h = jnp.maximum(jnp.dot(h, w1_ref[...], preferred_element_type=f32) + b1_ref[...], 0.0)
    h = jnp.maximum(jnp.dot(h, w2_ref[...], preferred_element_type=f32) + b2_ref[...], 0.0)
    mf = ie_mf_ref[...] * jnp.dot(sel, ue_mf_ref[...], preferred_element_type=f32)
    out_ref[...] = (jnp.dot(h, wam_ref[...], preferred_element_type=f32)
                    + jnp.dot(mf, waf_ref[...], preferred_element_type=f32)
                    + ba_ref[...])


def _tc_mlp(ie_mlp, ie_mf, ue_mlp, ue_mf, w0u, w0i, w1, w2, b0, b1, b2,
            wam, waf, ba):
    full = lambda shape: pl.BlockSpec(shape, lambda i: (0, 0))
    return pl.pallas_call(
        _tc_body,
        grid=(B // BB,),
        in_specs=[
            pl.BlockSpec((R, D), lambda i: (i, 0)),
            pl.BlockSpec((R, D), lambda i: (i, 0)),
            pl.BlockSpec((BB, D), lambda i: (i, 0)),
            pl.BlockSpec((BB, D), lambda i: (i, 0)),
            full((D, 64)),
            full((D, 64)),
            full((64, 32)),
            full((32, D)),
            full((1, 64)),
            full((1, 32)),
            full((1, D)),
            full((D, 1)),
            full((D, 1)),
            full((1, 1)),
        ],
        out_specs=pl.BlockSpec((R, 1), lambda i: (i, 0)),
        out_shape=jax.ShapeDtypeStruct((BL, 1), jnp.float32),
    )(ie_mlp, ie_mf, ue_mlp, ue_mf, w0u, w0i, w1, w2, b0, b1, b2, wam, waf, ba)


def kernel(user_indices, item_indices, emb_user_mlp, emb_item_mlp,
           emb_user_mf, emb_item_mf, W0, b0, W1, b1, W2, b2, Wa, ba):
    item_flat = _permute_idx(item_indices.reshape(-1).astype(jnp.int32))
    user_idx = _permute_idx(user_indices.astype(jnp.int32))
    g_imlp = _tc_repack(emb_item_mlp.T)
    g_imf = _tc_repack(emb_item_mf.T)
    g_umlp = _tc_repack(emb_user_mlp.T)
    g_umf = _tc_repack(emb_user_mf.T)
    ie_mlp, ie_mf, ue_mlp, ue_mf = _sc_gather(
        item_flat, user_idx, g_imlp, g_imf, g_umlp, g_umf)
    out = _tc_mlp(
        ie_mlp, ie_mf, ue_mlp, ue_mf,
        W0[:D], W0[D:], W1, W2,
        b0[None, :], b1[None, :], b2[None, :],
        Wa[:D], Wa[D:], ba[None, :])
    return out.reshape(B, 1, L)
